# baseline (speedup 1.0000x reference)
import functools
import math

import jax
import jax.numpy as jnp
from jax.experimental import pallas as pl
from jax.experimental.pallas import tpu as pltpu

_BLK = 128


def _fc_softmax_kernel(x_ref, v_ref, a_ref, o_ref, *, k_top):
    a = a_ref[...]
    m = jnp.max(a, axis=1, keepdims=True)
    e = jnp.exp(a - m)
    probs = e / jnp.sum(e, axis=1, keepdims=True)
    s = jnp.clip(k_top * probs, 0.0, 1.0)

    i = pl.program_id(0)
    vt = v_ref[...].T
    vts = vt * s
    wt = pltpu.roll(vts, 0, 1, stride=1, stride_axis=0)
    wt = pltpu.roll(wt, i * _BLK, 1)
    part = jax.lax.dot_general(
        x_ref[...], wt,
        dimension_numbers=(((1,), (0,)), ((), ())),
        preferred_element_type=jnp.float32,
        precision=jax.lax.Precision.HIGHEST,
    )

    @pl.when(i == 0)
    def _init():
        o_ref[...] = part

    @pl.when(i != 0)
    def _acc():
        o_ref[...] += part


def kernel(x, V, alpha):
    total, diag = V.shape
    batch, in_f = x.shape
    sparsity = 0.1
    k_top = math.ceil(int((1 - sparsity) * in_f * total) / diag)
    n_blk = diag // _BLK
    return pl.pallas_call(
        functools.partial(_fc_softmax_kernel, k_top=float(k_top)),
        grid=(n_blk,),
        in_specs=[
            pl.BlockSpec((batch, _BLK), lambda i: (0, i)),
            pl.BlockSpec((total, _BLK), lambda i: (0, i)),
            pl.BlockSpec((1, total), lambda i: (0, 0)),
        ],
        out_specs=pl.BlockSpec((batch, total), lambda i: (0, 0)),
        out_shape=jax.ShapeDtypeStruct((batch, total), jnp.float32),
    )(x, V, alpha.reshape(1, total))


# speedup vs baseline: 2.0617x; 2.0617x over previous
import math

import jax
import jax.numpy as jnp
from jax.experimental import pallas as pl
from jax.experimental.pallas import tpu as pltpu


def _fc_softmax_kernel(x_ref, v_ref, a_ref, o_ref, *, k_top):
    a = a_ref[...]
    m = jnp.max(a, axis=1, keepdims=True)
    e = jnp.exp(a - m)
    probs = e / jnp.sum(e, axis=1, keepdims=True)
    s = jnp.clip(k_top * probs, 0.0, 1.0)

    vt = v_ref[...].T
    vts = vt * s
    wt = pltpu.roll(vts, 0, 1, stride=1, stride_axis=0)
    o_ref[...] = jax.lax.dot_general(
        x_ref[...], wt,
        dimension_numbers=(((1,), (0,)), ((), ())),
        preferred_element_type=jnp.float32,
        precision=jax.lax.Precision.DEFAULT,
    )


def kernel(x, V, alpha):
    total, diag = V.shape
    batch, in_f = x.shape
    sparsity = 0.1
    k_top = math.ceil(int((1 - sparsity) * in_f * total) / diag)
    return pl.pallas_call(
        lambda x_ref, v_ref, a_ref, o_ref: _fc_softmax_kernel(
            x_ref, v_ref, a_ref, o_ref, k_top=float(k_top)),
        out_shape=jax.ShapeDtypeStruct((batch, total), jnp.float32),
    )(x, V, alpha.reshape(1, total))


# speedup vs baseline: 2.4052x; 1.1666x over previous
import math

import jax
import jax.numpy as jnp
from jax.experimental import pallas as pl
from jax.experimental.pallas import tpu as pltpu


def _fc_softmax_kernel(x_ref, v_ref, a_ref, o_ref, *, k_top):
    a = a_ref[...]
    m = jnp.max(a, axis=1, keepdims=True)
    e = jnp.exp(a - m)
    probs = e / jnp.sum(e, axis=1, keepdims=True)
    s = jnp.clip(k_top * probs, 0.0, 1.0)

    vb = v_ref[...].astype(jnp.bfloat16)
    vt = vb.T
    vts = vt * s.astype(jnp.bfloat16)
    wt = pltpu.roll(vts, 0, 1, stride=1, stride_axis=0)
    o_ref[...] = jax.lax.dot_general(
        x_ref[...], wt,
        dimension_numbers=(((1,), (0,)), ((), ())),
        preferred_element_type=jnp.float32,
        precision=jax.lax.Precision.DEFAULT,
    )


def kernel(x, V, alpha):
    total, diag = V.shape
    batch, in_f = x.shape
    sparsity = 0.1
    k_top = math.ceil(int((1 - sparsity) * in_f * total) / diag)
    return pl.pallas_call(
        lambda x_ref, v_ref, a_ref, o_ref: _fc_softmax_kernel(
            x_ref, v_ref, a_ref, o_ref, k_top=float(k_top)),
        out_shape=jax.ShapeDtypeStruct((batch, total), jnp.float32),
    )(x, V, alpha.reshape(1, total))
